# baseline (speedup 1.0000x reference)
import numpy as np
import jax
import jax.numpy as jnp
from jax import lax
from jax.experimental import pallas as pl

LOG2 = float(np.log(2.0))
CUTOFF = 5.0

B, A, N = 2, 512, 64
CB, NF, NSB = 64, 64, 50
BA = 64
ROWS = BA * N
F32 = jnp.float32
BF = jnp.bfloat16


def _ssp(x):
    return jnp.maximum(x, 0.0) + jnp.log(1.0 + jnp.exp(-jnp.abs(x))) - LOG2


def _ssp_fast(x):
    return jnp.log(0.5 + 0.5 * jnp.exp(x))


_COS_C = [float(c) for c in
          (1.0, -1.0 / 2, 1.0 / 24, -1.0 / 720, 1.0 / 40320,
           -1.0 / 3628800, 1.0 / 479001600, -1.0 / 87178291200)]


def _cos_poly(x):
    u = x * x
    acc = jnp.full_like(x, _COS_C[-1])
    for c in _COS_C[-2::-1]:
        acc = acc * u + c
    return acc


def _kernel_a(xi_ref, f_ref, r_ref, mask_ref, c0_ref, c1_ref, c2_ref, nbh_ref,
              wf1_ref, bf1_ref, wf2_ref, bf2_ref, win2f_ref, wf2o_ref, bf2o_ref,
              wa1_ref, ba1_ref, wa2_ref, ba2_ref, wpe1_ref, bpe1_ref, wpe2_ref,
              bpe2_ref, tself_ref, ohn_ref, bd_ref, msel_ref, b64_ref,
              vi_ref, vij_ref, vikt_ref):
    y = jnp.dot(xi_ref[0].astype(BF), win2f_ref[...].astype(BF),
                preferred_element_type=F32).astype(BF)
    for half in range(2):
        _kernel_a_half(half, y, f_ref, r_ref, mask_ref, c0_ref, c1_ref,
                       c2_ref, nbh_ref, wf1_ref, bf1_ref, wf2_ref, bf2_ref,
                       wf2o_ref, bf2o_ref, wa1_ref, ba1_ref, wa2_ref, ba2_ref,
                       wpe1_ref, bpe1_ref, wpe2_ref, bpe2_ref,
                       tself_ref, ohn_ref, bd_ref, msel_ref, b64_ref,
                       vi_ref, vij_ref, vikt_ref)


HA = BA // 2
HROWS = HA * N


def _hexpand(tself, ohn, bd, scalars):
    s = jnp.concatenate(scalars, axis=1).astype(BF)
    x = jnp.dot(tself, s, preferred_element_type=F32)
    sel = (x * ohn).astype(BF)
    out = jnp.dot(sel, bd, preferred_element_type=F32)
    return [out[:, i * N:(i + 1) * N] for i in range(len(scalars))]


def _honehot(nbh_b, b64):
    return jnp.concatenate(
        [(nbh_b == b64 + (64.0 * k)).astype(BF) for k in range(A // N)],
        axis=1)


def _kernel_a_half(half, y, f_ref, r_ref, mask_ref, c0_ref, c1_ref, c2_ref,
                   nbh_ref, wf1_ref, bf1_ref, wf2_ref, bf2_ref, wf2o_ref,
                   bf2o_ref, wa1_ref, ba1_ref, wa2_ref, ba2_ref, wpe1_ref,
                   bpe1_ref, wpe2_ref, bpe2_ref, tself_ref, ohn_ref, bd_ref,
                   msel_ref, b64_ref, vi_ref, vij_ref, vikt_ref):
    a0 = half * HA
    f = f_ref[0, a0:a0 + HA].reshape(HROWS, NSB)
    h = _ssp_fast(jnp.dot(f.astype(BF), wf1_ref[...].astype(BF),
                     preferred_element_type=F32) + bf1_ref[...])
    wfilt = jnp.dot(h.astype(BF), wf2_ref[...].astype(BF),
                    preferred_element_type=F32) + bf2_ref[...]

    r = r_ref[0, a0:a0 + HA]
    x = jnp.minimum(r * (np.pi / CUTOFF), float(np.pi))
    cc = 0.5 * (_cos_poly(x) + 1.0) * (r < CUTOFF).astype(F32)

    nbh_b, cc_b = _hexpand(tself_ref[...], ohn_ref[...], bd_ref[...],
                           (nbh_ref[0, a0:a0 + HA].astype(F32) - 256.0, cc))
    oh = _honehot(nbh_b, b64_ref[...])

    yg = jnp.dot(oh, y, preferred_element_type=F32)
    yv = yg * wfilt * cc_b
    v = _ssp_fast(jnp.dot(yv.astype(BF), wf2o_ref[...].astype(BF),
                     preferred_element_type=F32) + bf2o_ref[...])

    vb = v.astype(BF)
    z = _ssp_fast(jnp.dot(vb, wpe1_ref[...].astype(BF),
                     preferred_element_type=F32) + bpe1_ref[...])
    w2 = jnp.dot(z.astype(BF), wpe2_ref[...].astype(BF),
                 preferred_element_type=F32) + bpe2_ref[...]
    vij = w2[:, :NF]
    vik = w2[:, NF:]
    vij_ref[0, a0:a0 + HA] = vij.astype(BF).reshape(HA, N, NF)

    msel = msel_ref[...]
    mask = mask_ref[0, a0:a0 + HA]
    mm = (msel * jnp.tile(mask, (1, HA))).astype(BF)
    vsum = jnp.dot(mm, vb, preferred_element_type=F32)
    vi = jnp.dot(_ssp(jnp.dot(vsum.astype(BF), wa1_ref[...].astype(BF),
                              preferred_element_type=F32) + ba1_ref[...]).astype(BF),
                 wa2_ref[...].astype(BF), preferred_element_type=F32) + ba2_ref[...]
    vi_ref[0, a0:a0 + HA] = vi

    vikb = vik.astype(BF)
    agg = [jnp.dot((msel * jnp.tile(mask * cref[0, a0:a0 + HA], (1, HA))).astype(BF),
                   vikb, preferred_element_type=F32)
           for cref in (c0_ref, c1_ref, c2_ref)]
    vikt_ref[0, a0:a0 + HA] = jnp.concatenate(agg, axis=1)


NB = 16


def _kernel_b(vij_ref, vikt_ref, nbht_ref, cost_ref, out_ref):
    e0 = lax.broadcasted_iota(jnp.int32, (A, A), 0)
    eye = (e0 == lax.broadcasted_iota(jnp.int32, (A, A), 1)).astype(BF)
    cdim = (((0,), (0,)), ((), ()))
    viktT = lax.dot_general(vikt_ref[0].astype(BF), eye, cdim,
                            preferred_element_type=F32)
    viktTb = viktT.astype(BF)
    for n in range(NB):
        vijT = lax.dot_general(vij_ref[0, :, n, :], eye, cdim,
                               preferred_element_type=F32)
        ohT = (e0 == nbht_ref[0, n:n + 1, :]).astype(BF)
        vjlT = jnp.dot(viktTb, ohT, preferred_element_type=F32)
        for d in range(3):
            lo = d * CB
            cosd = cost_ref[0, d, n:n + 1, :]
            out_ref[0, n, d] = (vijT * cosd + viktT[lo:lo + CB]
                                + vjlT[lo:lo + CB])


def _block(*shape):
    def im(b, i):
        return (b,) + (0,) * (len(shape) - 1)
    return pl.BlockSpec(shape, im)


def _cblock(*shape):
    def im(b, i):
        return (0,) * len(shape)
    return pl.BlockSpec(shape, im)


def _ablock(spec_shape, pos=1):
    def im(b, i):
        idx = [0] * len(spec_shape)
        idx[0] = b
        idx[pos] = i
        return tuple(idx)
    return pl.BlockSpec(spec_shape, im)


def kernel(xi, r_ij, cos_ij, neighbors, neighbor_mask, f_ij, Wf1, bf1, Wf2, bf2,
           Win2f, Wf2out, bf2out, Wa1, ba1, Wa2, ba2, Wp1, bp1, Wp2, bp2,
           We1, be1, We2, be2):
    grid = (B, A // BA)
    c0, c1, c2 = cos_ij[..., 0], cos_ij[..., 1], cos_ij[..., 2]
    wpe1 = jnp.concatenate([Wp1.T, We1.T], axis=1)
    bpe1 = jnp.concatenate([bp1, be1])[None]
    z64 = jnp.zeros((NF, CB), F32)
    wpe2 = jnp.concatenate(
        [jnp.concatenate([Wp2.T, z64], axis=1),
         jnp.concatenate([z64, We2.T], axis=1)], axis=0)
    bpe2 = jnp.concatenate([bp2, be2])[None]
    wspecs = [
        _block(NSB, NF), _block(1, NF), _block(NF, NF), _block(1, NF),
        _block(CB, NF), _block(NF, NF), _block(1, NF),
        _block(NF, NF), _block(1, NF), _block(NF, CB), _block(1, CB),
        _block(NF, 2 * NF), _block(1, 2 * NF),
        _block(2 * NF, 2 * NF), _block(1, 2 * NF),
    ]
    wargs = [Wf1.T, bf1[None], Wf2.T, bf2[None], Win2f.T, Wf2out.T, bf2out[None],
             Wa1.T, ba1[None], Wa2.T, ba2[None], wpe1, bpe1, wpe2, bpe2]

    arr = jnp.arange(HROWS)
    aha = jnp.arange(HA)
    tself_c = (arr[:, None] // N == aha[None, :]).astype(BF)
    j2 = jnp.arange(2 * N)
    ohn_c = (arr[:, None] % N == j2[None, :] % N).astype(F32)
    bd_c = (j2[:, None] // N == j2[None, :] // N).astype(BF)
    msel_c = (aha[:, None] == arr[None, :] // N).astype(F32)
    b64_c = (jnp.arange(N).astype(F32) - 256.0)[None]
    cspecs = [_cblock(HROWS, HA), _cblock(HROWS, 2 * N), _cblock(2 * N, 2 * N),
              _cblock(HA, HROWS), _cblock(1, N)]
    cargs = [tself_c, ohn_c, bd_c, msel_c, b64_c]

    vi, vij, vikt = pl.pallas_call(
        _kernel_a,
        grid=grid,
        in_specs=[
            _block(1, A, CB),
            _ablock((1, BA, N, NSB)),
            _ablock((1, BA, N)),
            _ablock((1, BA, N)),
            _ablock((1, BA, N)),
            _ablock((1, BA, N)),
            _ablock((1, BA, N)),
            _ablock((1, BA, N)),
        ] + wspecs + cspecs,
        out_specs=[
            _ablock((1, BA, CB)),
            _ablock((1, BA, N, NF)),
            _ablock((1, BA, 3 * CB)),
        ],
        out_shape=[
            jax.ShapeDtypeStruct((B, A, CB), F32),
            jax.ShapeDtypeStruct((B, A, N, NF), BF),
            jax.ShapeDtypeStruct((B, A, 3 * CB), F32),
        ],
    )(xi, f_ij, r_ij, neighbor_mask, c0, c1, c2, neighbors, *wargs, *cargs)

    nbh_t = jnp.swapaxes(neighbors, 1, 2)
    cos_t = jnp.transpose(cos_ij, (0, 3, 2, 1))
    (vp,) = pl.pallas_call(
        _kernel_b,
        grid=(B, N // NB),
        in_specs=[
            _ablock((1, A, NB, NF), 2),
            _block(1, A, 3 * CB),
            _ablock((1, NB, A), 1),
            _ablock((1, 3, NB, A), 2),
        ],
        out_specs=[
            _ablock((1, NB, 3, CB, A), 1),
        ],
        out_shape=[
            jax.ShapeDtypeStruct((B, N, 3, CB, A), F32),
        ],
    )(vij, vikt, nbh_t, cos_t)

    V = jnp.transpose(vp, (0, 4, 1, 3, 2))
    return vi, V


# speedup vs baseline: 1.0064x; 1.0064x over previous
import numpy as np
import jax
import jax.numpy as jnp
from jax import lax
from jax.experimental import pallas as pl

LOG2 = float(np.log(2.0))
CUTOFF = 5.0

B, A, N = 2, 512, 64
CB, NF, NSB = 64, 64, 50
BA = 64
ROWS = BA * N
F32 = jnp.float32
BF = jnp.bfloat16


def _ssp(x):
    return jnp.maximum(x, 0.0) + jnp.log(1.0 + jnp.exp(-jnp.abs(x))) - LOG2


def _ssp_fast(x):
    return jnp.log(0.5 + 0.5 * jnp.exp(x))


_COS_C = [float(c) for c in
          (1.0, -1.0 / 2, 1.0 / 24, -1.0 / 720, 1.0 / 40320,
           -1.0 / 3628800, 1.0 / 479001600, -1.0 / 87178291200)]


def _cos_poly(x):
    u = x * x
    acc = jnp.full_like(x, _COS_C[-1])
    for c in _COS_C[-2::-1]:
        acc = acc * u + c
    return acc


def _kernel_a(xi_ref, f_ref, r_ref, mask_ref, c0_ref, c1_ref, c2_ref, nbh_ref,
              wf1_ref, bf1_ref, wf2_ref, bf2_ref, win2f_ref, wf2o_ref, bf2o_ref,
              wa1_ref, ba1_ref, wa2_ref, ba2_ref, wpe1_ref, bpe1_ref, wpe2_ref,
              bpe2_ref, vi_ref, vij_ref, vikt_ref):
    y = jnp.dot(xi_ref[0].astype(BF), win2f_ref[...].astype(BF),
                preferred_element_type=F32).astype(BF)
    for half in range(2):
        _kernel_a_half(half, y, f_ref, r_ref, mask_ref, c0_ref, c1_ref,
                       c2_ref, nbh_ref, wf1_ref, bf1_ref, wf2_ref, bf2_ref,
                       wf2o_ref, bf2o_ref, wa1_ref, ba1_ref, wa2_ref, ba2_ref,
                       wpe1_ref, bpe1_ref, wpe2_ref, bpe2_ref,
                       vi_ref, vij_ref, vikt_ref)


HA = BA // 2
HROWS = HA * N


def _hselectors(k):
    i0 = lax.broadcasted_iota(jnp.int32, (HROWS, HA), 0) // N
    i1 = lax.broadcasted_iota(jnp.int32, (HROWS, HA), 1)
    tself = (i0 == i1).astype(BF)
    j0 = lax.broadcasted_iota(jnp.int32, (HROWS, k * N), 0) % N
    j1 = lax.broadcasted_iota(jnp.int32, (HROWS, k * N), 1) % N
    ohn = (j0 == j1).astype(F32)
    b0 = lax.broadcasted_iota(jnp.int32, (k * N, k * N), 0) // N
    b1 = lax.broadcasted_iota(jnp.int32, (k * N, k * N), 1) // N
    bd = (b0 == b1).astype(BF)
    return tself, ohn, bd


def _hexpand(tself, ohn, bd, scalars):
    s = jnp.concatenate(scalars, axis=1).astype(BF)
    x = jnp.dot(tself, s, preferred_element_type=F32)
    sel = (x * ohn).astype(BF)
    out = jnp.dot(sel, bd, preferred_element_type=F32)
    return [out[:, i * N:(i + 1) * N] for i in range(len(scalars))]


def _honehot(nbh_b):
    base = lax.broadcasted_iota(jnp.int32, (HROWS, N), 1).astype(F32) - 256.0
    return jnp.concatenate(
        [(nbh_b == base + (64.0 * k)).astype(BF) for k in range(A // N)],
        axis=1)


def _kernel_a_half(half, y, f_ref, r_ref, mask_ref, c0_ref, c1_ref, c2_ref,
                   nbh_ref, wf1_ref, bf1_ref, wf2_ref, bf2_ref, wf2o_ref,
                   bf2o_ref, wa1_ref, ba1_ref, wa2_ref, ba2_ref, wpe1_ref,
                   bpe1_ref, wpe2_ref, bpe2_ref, vi_ref, vij_ref, vikt_ref):
    a0 = half * HA
    f = f_ref[0, a0:a0 + HA].reshape(HROWS, NSB)
    h = _ssp_fast(jnp.dot(f.astype(BF), wf1_ref[...].astype(BF),
                     preferred_element_type=F32) + bf1_ref[...])
    wfilt = jnp.dot(h.astype(BF), wf2_ref[...].astype(BF),
                    preferred_element_type=F32) + bf2_ref[...]

    r = r_ref[0, a0:a0 + HA]
    x = jnp.minimum(r * (np.pi / CUTOFF), float(np.pi))
    cc = 0.5 * (_cos_poly(x) + 1.0) * (r < CUTOFF).astype(F32)

    tself, ohn, bd = _hselectors(2)
    nbh_b, cc_b = _hexpand(tself, ohn, bd,
                           (nbh_ref[0, a0:a0 + HA].astype(F32) - 256.0, cc))
    oh = _honehot(nbh_b)

    yg = jnp.dot(oh, y, preferred_element_type=F32)
    yv = yg * wfilt * cc_b
    v = _ssp_fast(jnp.dot(yv.astype(BF), wf2o_ref[...].astype(BF),
                     preferred_element_type=F32) + bf2o_ref[...])

    vb = v.astype(BF)
    z = _ssp_fast(jnp.dot(vb, wpe1_ref[...].astype(BF),
                     preferred_element_type=F32) + bpe1_ref[...])
    w2 = jnp.dot(z.astype(BF), wpe2_ref[...].astype(BF),
                 preferred_element_type=F32) + bpe2_ref[...]
    vij = w2[:, :NF]
    vik = w2[:, NF:]
    vij_ref[0, a0:a0 + HA] = vij.astype(BF).reshape(HA, N, NF)

    m0 = lax.broadcasted_iota(jnp.int32, (HA, HROWS), 0)
    m1 = lax.broadcasted_iota(jnp.int32, (HA, HROWS), 1) // N
    msel = (m0 == m1).astype(F32)
    mask = mask_ref[0, a0:a0 + HA]
    mm = (msel * jnp.tile(mask, (1, HA))).astype(BF)
    vsum = jnp.dot(mm, vb, preferred_element_type=F32)
    vi = jnp.dot(_ssp(jnp.dot(vsum.astype(BF), wa1_ref[...].astype(BF),
                              preferred_element_type=F32) + ba1_ref[...]).astype(BF),
                 wa2_ref[...].astype(BF), preferred_element_type=F32) + ba2_ref[...]
    vi_ref[0, a0:a0 + HA] = vi

    vikb = vik.astype(BF)
    agg = [jnp.dot((msel * jnp.tile(mask * cref[0, a0:a0 + HA], (1, HA))).astype(BF),
                   vikb, preferred_element_type=F32)
           for cref in (c0_ref, c1_ref, c2_ref)]
    vikt_ref[0, a0:a0 + HA] = jnp.concatenate(agg, axis=1)


NB = 8


def _kernel_b(vij_ref, vikt_ref, nbht_ref, cost_ref, out_ref):
    e0 = lax.broadcasted_iota(jnp.int32, (A, A), 0)
    eye = (e0 == lax.broadcasted_iota(jnp.int32, (A, A), 1)).astype(BF)
    cdim = (((0,), (0,)), ((), ()))
    viktT = lax.dot_general(vikt_ref[0].astype(BF), eye, cdim,
                            preferred_element_type=F32)
    viktTb = viktT.astype(BF)
    for n in range(NB):
        vijT = lax.dot_general(vij_ref[0, :, n, :], eye, cdim,
                               preferred_element_type=F32)
        ohT = (e0 == nbht_ref[0, n:n + 1, :]).astype(BF)
        vjlT = jnp.dot(viktTb, ohT, preferred_element_type=F32)
        for d in range(3):
            lo = d * CB
            cosd = cost_ref[0, d, n:n + 1, :]
            out_ref[0, n, d] = (vijT * cosd + viktT[lo:lo + CB]
                                + vjlT[lo:lo + CB])


def _block(*shape):
    def im(b, i):
        return (b,) + (0,) * (len(shape) - 1)
    return pl.BlockSpec(shape, im)


def _ablock(spec_shape, pos=1):
    def im(b, i):
        idx = [0] * len(spec_shape)
        idx[0] = b
        idx[pos] = i
        return tuple(idx)
    return pl.BlockSpec(spec_shape, im)


def kernel(xi, r_ij, cos_ij, neighbors, neighbor_mask, f_ij, Wf1, bf1, Wf2, bf2,
           Win2f, Wf2out, bf2out, Wa1, ba1, Wa2, ba2, Wp1, bp1, Wp2, bp2,
           We1, be1, We2, be2):
    grid = (B, A // BA)
    c0, c1, c2 = cos_ij[..., 0], cos_ij[..., 1], cos_ij[..., 2]
    wpe1 = jnp.concatenate([Wp1.T, We1.T], axis=1)
    bpe1 = jnp.concatenate([bp1, be1])[None]
    z64 = jnp.zeros((NF, CB), F32)
    wpe2 = jnp.concatenate(
        [jnp.concatenate([Wp2.T, z64], axis=1),
         jnp.concatenate([z64, We2.T], axis=1)], axis=0)
    bpe2 = jnp.concatenate([bp2, be2])[None]
    wspecs = [
        _block(NSB, NF), _block(1, NF), _block(NF, NF), _block(1, NF),
        _block(CB, NF), _block(NF, NF), _block(1, NF),
        _block(NF, NF), _block(1, NF), _block(NF, CB), _block(1, CB),
        _block(NF, 2 * NF), _block(1, 2 * NF),
        _block(2 * NF, 2 * NF), _block(1, 2 * NF),
    ]
    wargs = [Wf1.T, bf1[None], Wf2.T, bf2[None], Win2f.T, Wf2out.T, bf2out[None],
             Wa1.T, ba1[None], Wa2.T, ba2[None], wpe1, bpe1, wpe2, bpe2]


    vi, vij, vikt = pl.pallas_call(
        _kernel_a,
        grid=grid,
        in_specs=[
            _block(1, A, CB),
            _ablock((1, BA, N, NSB)),
            _ablock((1, BA, N)),
            _ablock((1, BA, N)),
            _ablock((1, BA, N)),
            _ablock((1, BA, N)),
            _ablock((1, BA, N)),
            _ablock((1, BA, N)),
        ] + wspecs,
        out_specs=[
            _ablock((1, BA, CB)),
            _ablock((1, BA, N, NF)),
            _ablock((1, BA, 3 * CB)),
        ],
        out_shape=[
            jax.ShapeDtypeStruct((B, A, CB), F32),
            jax.ShapeDtypeStruct((B, A, N, NF), BF),
            jax.ShapeDtypeStruct((B, A, 3 * CB), F32),
        ],
    )(xi, f_ij, r_ij, neighbor_mask, c0, c1, c2, neighbors, *wargs)

    nbh_t = jnp.swapaxes(neighbors, 1, 2)
    cos_t = jnp.transpose(cos_ij, (0, 3, 2, 1))
    (vp,) = pl.pallas_call(
        _kernel_b,
        grid=(B, N // NB),
        in_specs=[
            _ablock((1, A, NB, NF), 2),
            _block(1, A, 3 * CB),
            _ablock((1, NB, A), 1),
            _ablock((1, 3, NB, A), 2),
        ],
        out_specs=[
            _ablock((1, NB, 3, CB, A), 1),
        ],
        out_shape=[
            jax.ShapeDtypeStruct((B, N, 3, CB, A), F32),
        ],
    )(vij, vikt, nbh_t, cos_t)

    V = jnp.transpose(vp, (0, 4, 1, 3, 2))
    return vi, V
